# baseline (speedup 1.0000x reference)
import functools

import jax
import jax.numpy as jnp
from jax import lax
from jax.experimental import pallas as pl
from jax.experimental.pallas import tpu as pltpu
from jax.experimental.pallas import tpu_sc as plsc

K = 1024
D = 64
B = 8192
BM = 4096
NB = B // BM


def _assign_body(xt_ref, ct_ref, cc_ref, out_ref):
    xt = xt_ref[...]
    ct = ct_ref[...]
    cc = cc_ref[...]
    xx = jnp.sum(xt * xt, axis=0, keepdims=True)
    xc = lax.dot_general(
        ct, xt,
        dimension_numbers=(((0,), (0,)), ((), ())),
        preferred_element_type=jnp.float32,
    )
    dists = xx - 2.0 * xc + cc
    minval = jnp.min(dists, axis=0, keepdims=True)
    ids = lax.broadcasted_iota(jnp.int32, dists.shape, 0).astype(jnp.float32)
    amin = jnp.min(jnp.where(dists == minval, ids, float(K)), axis=0)
    out_ref[...] = amin.astype(jnp.int32)


def _assign(xt, ct, cc):
    return pl.pallas_call(
        _assign_body,
        grid=(NB,),
        in_specs=[
            pl.BlockSpec((D, BM), lambda i: (0, i)),
            pl.BlockSpec((D, K), lambda i: (0, 0)),
            pl.BlockSpec((K, 1), lambda i: (0, 0)),
        ],
        out_specs=pl.BlockSpec((BM,), lambda i: (i,)),
        out_shape=jax.ShapeDtypeStruct((B,), jnp.int32),
    )(xt, ct, cc)


def _make_sc_gather():
    info = plsc.get_sparse_core_info()
    nw = info.num_cores * info.num_subcores
    b_per_w = B // nw
    mesh = plsc.VectorSubcoreMesh(core_axis_name="c", subcore_axis_name="s")

    @functools.partial(
        pl.kernel, mesh=mesh,
        compiler_params=pltpu.CompilerParams(use_tc_tiling_on_sc=False),
        out_type=jax.ShapeDtypeStruct((B, D), jnp.float32),
        scratch_types=[
            pltpu.VMEM((b_per_w,), jnp.int32),
            pltpu.VMEM((b_per_w, D), jnp.float32),
            pltpu.SemaphoreType.DMA,
        ],
    )
    def gather_k(table_hbm, idx_hbm, out_hbm, idx_v, rows_v, sem):
        wid = lax.axis_index("s") * info.num_cores + lax.axis_index("c")
        base = wid * b_per_w
        pltpu.sync_copy(idx_hbm.at[pl.ds(base, b_per_w)], idx_v)
        pltpu.async_copy(table_hbm.at[idx_v], rows_v, sem).wait()
        pltpu.sync_copy(rows_v, out_hbm.at[pl.ds(base, b_per_w)])

    return gather_k


_sc_gather = _make_sc_gather()


def _make_sc_gather_t():
    info = plsc.get_sparse_core_info()
    nw = info.num_cores * info.num_subcores
    f_per_w = D // nw
    mesh = plsc.VectorSubcoreMesh(core_axis_name="c", subcore_axis_name="s")

    @functools.partial(
        pl.kernel, mesh=mesh,
        compiler_params=pltpu.CompilerParams(
            use_tc_tiling_on_sc=False, needs_layout_passes=False),
        out_type=jax.ShapeDtypeStruct((D, B), jnp.float32),
        scratch_types=[
            pltpu.VMEM((B,), jnp.int32),
            pltpu.VMEM((f_per_w * K,), jnp.float32),
            pltpu.VMEM((f_per_w, B), jnp.float32),
        ],
    )
    def gather_t(tablet_hbm, idx_hbm, out_hbm, idx_v, tbl_v, out_v):
        wid = lax.axis_index("s") * info.num_cores + lax.axis_index("c")
        base = wid * f_per_w
        pltpu.sync_copy(tablet_hbm.at[pl.ds(base * K, f_per_w * K)], tbl_v)
        pltpu.sync_copy(idx_hbm, idx_v)
        kk = jnp.full((16,), K, jnp.int32)

        @plsc.parallel_loop(0, B // 16, unroll=8)
        def body(g):
            o = g * 16
            iv = idx_v[pl.ds(o, 16)]
            v0 = plsc.load_gather(tbl_v, [iv])
            v1 = plsc.load_gather(tbl_v, [iv + kk])
            out_v[0, pl.ds(o, 16)] = v0
            out_v[1, pl.ds(o, 16)] = v1
        pltpu.sync_copy(out_v, out_hbm.at[pl.ds(base, f_per_w)])

    return gather_t


_sc_gather_t = _make_sc_gather_t()


def kernel(x, centroids):
    cc = jnp.sum(centroids * centroids, axis=1, keepdims=True)
    assign = _assign(x.T, centroids.T, cc)
    gathered = _sc_gather_t(centroids.T.reshape(D * K), assign).T
    return (assign, gathered)


# speedup vs baseline: 1.0100x; 1.0100x over previous
import functools

import jax
import jax.numpy as jnp
from jax import lax
from jax.experimental import pallas as pl
from jax.experimental.pallas import tpu as pltpu
from jax.experimental.pallas import tpu_sc as plsc

K = 1024
D = 64
B = 8192
BM = 2048
NB = B // BM


def _assign_body(xt_ref, ct_ref, cc_ref, out_ref):
    xt = xt_ref[...]
    ct = ct_ref[...]
    cc = cc_ref[...]
    xx = jnp.sum(xt * xt, axis=0, keepdims=True)
    xc = lax.dot_general(
        ct, xt,
        dimension_numbers=(((0,), (0,)), ((), ())),
        preferred_element_type=jnp.float32,
    )
    dists = xx - 2.0 * xc + cc
    minval = jnp.min(dists, axis=0, keepdims=True)
    ids = lax.broadcasted_iota(jnp.int32, dists.shape, 0).astype(jnp.float32)
    amin = jnp.min(jnp.where(dists == minval, ids, float(K)), axis=0)
    out_ref[...] = amin.astype(jnp.int32)


def _assign(xt, ct, cc):
    return pl.pallas_call(
        _assign_body,
        grid=(NB,),
        in_specs=[
            pl.BlockSpec((D, BM), lambda i: (0, i)),
            pl.BlockSpec((D, K), lambda i: (0, 0)),
            pl.BlockSpec((K, 1), lambda i: (0, 0)),
        ],
        out_specs=pl.BlockSpec((BM,), lambda i: (i,)),
        out_shape=jax.ShapeDtypeStruct((B,), jnp.int32),
    )(xt, ct, cc)


def _make_sc_gather():
    info = plsc.get_sparse_core_info()
    nw = info.num_cores * info.num_subcores
    b_per_w = B // nw
    mesh = plsc.VectorSubcoreMesh(core_axis_name="c", subcore_axis_name="s")

    @functools.partial(
        pl.kernel, mesh=mesh,
        compiler_params=pltpu.CompilerParams(use_tc_tiling_on_sc=False),
        out_type=jax.ShapeDtypeStruct((B, D), jnp.float32),
        scratch_types=[
            pltpu.VMEM((b_per_w,), jnp.int32),
            pltpu.VMEM((b_per_w, D), jnp.float32),
            pltpu.SemaphoreType.DMA,
        ],
    )
    def gather_k(table_hbm, idx_hbm, out_hbm, idx_v, rows_v, sem):
        wid = lax.axis_index("s") * info.num_cores + lax.axis_index("c")
        base = wid * b_per_w
        pltpu.sync_copy(idx_hbm.at[pl.ds(base, b_per_w)], idx_v)
        pltpu.async_copy(table_hbm.at[idx_v], rows_v, sem).wait()
        pltpu.sync_copy(rows_v, out_hbm.at[pl.ds(base, b_per_w)])

    return gather_k


_sc_gather = _make_sc_gather()


def _make_sc_gather_t():
    info = plsc.get_sparse_core_info()
    nw = info.num_cores * info.num_subcores
    f_per_w = D // nw
    mesh = plsc.VectorSubcoreMesh(core_axis_name="c", subcore_axis_name="s")

    @functools.partial(
        pl.kernel, mesh=mesh,
        compiler_params=pltpu.CompilerParams(
            use_tc_tiling_on_sc=False, needs_layout_passes=False),
        out_type=jax.ShapeDtypeStruct((D, B), jnp.float32),
        scratch_types=[
            pltpu.VMEM((B,), jnp.int32),
            pltpu.VMEM((f_per_w * K,), jnp.float32),
            pltpu.VMEM((f_per_w, B), jnp.float32),
        ],
    )
    def gather_t(tablet_hbm, idx_hbm, out_hbm, idx_v, tbl_v, out_v):
        wid = lax.axis_index("s") * info.num_cores + lax.axis_index("c")
        base = wid * f_per_w
        pltpu.sync_copy(tablet_hbm.at[pl.ds(base * K, f_per_w * K)], tbl_v)
        pltpu.sync_copy(idx_hbm, idx_v)
        kk = jnp.full((16,), K, jnp.int32)

        @plsc.parallel_loop(0, B // 16, unroll=16)
        def body(g):
            o = g * 16
            iv = idx_v[pl.ds(o, 16)]
            v0 = plsc.load_gather(tbl_v, [iv])
            v1 = plsc.load_gather(tbl_v, [iv + kk])
            out_v[0, pl.ds(o, 16)] = v0
            out_v[1, pl.ds(o, 16)] = v1
        pltpu.sync_copy(out_v, out_hbm.at[pl.ds(base, f_per_w)])

    return gather_t


_sc_gather_t = _make_sc_gather_t()


def kernel(x, centroids):
    cc = jnp.sum(centroids * centroids, axis=1, keepdims=True)
    assign = _assign(x.T, centroids.T, cc)
    gathered = _sc_gather_t(centroids.T.reshape(D * K), assign).T
    return (assign, gathered)


# speedup vs baseline: 1.0458x; 1.0355x over previous
import functools

import jax
import jax.numpy as jnp
from jax import lax
from jax.experimental import pallas as pl
from jax.experimental.pallas import tpu as pltpu
from jax.experimental.pallas import tpu_sc as plsc

K = 1024
D = 64
B = 8192
BM = 2048
NB = B // BM


def _assign_body(xt_ref, ct_ref, out_ref):
    xt = xt_ref[...]
    ct = ct_ref[...]
    cc = jnp.sum(ct * ct, axis=0, keepdims=True).T
    xx = jnp.sum(xt * xt, axis=0, keepdims=True)
    xc = lax.dot_general(
        ct, xt,
        dimension_numbers=(((0,), (0,)), ((), ())),
        preferred_element_type=jnp.float32,
    )
    dists = xx - 2.0 * xc + cc
    minval = jnp.min(dists, axis=0, keepdims=True)
    ids = lax.broadcasted_iota(jnp.int32, dists.shape, 0).astype(jnp.float32)
    amin = jnp.min(jnp.where(dists == minval, ids, float(K)), axis=0)
    out_ref[...] = amin.astype(jnp.int32)


def _assign(xt, ct):
    return pl.pallas_call(
        _assign_body,
        grid=(NB,),
        in_specs=[
            pl.BlockSpec((D, BM), lambda i: (0, i)),
            pl.BlockSpec((D, K), lambda i: (0, 0)),
        ],
        out_specs=pl.BlockSpec((BM,), lambda i: (i,)),
        out_shape=jax.ShapeDtypeStruct((B,), jnp.int32),
    )(xt, ct)


def _make_sc_gather():
    info = plsc.get_sparse_core_info()
    nw = info.num_cores * info.num_subcores
    b_per_w = B // nw
    mesh = plsc.VectorSubcoreMesh(core_axis_name="c", subcore_axis_name="s")

    @functools.partial(
        pl.kernel, mesh=mesh,
        compiler_params=pltpu.CompilerParams(use_tc_tiling_on_sc=False),
        out_type=jax.ShapeDtypeStruct((B, D), jnp.float32),
        scratch_types=[
            pltpu.VMEM((b_per_w,), jnp.int32),
            pltpu.VMEM((b_per_w, D), jnp.float32),
            pltpu.SemaphoreType.DMA,
        ],
    )
    def gather_k(table_hbm, idx_hbm, out_hbm, idx_v, rows_v, sem):
        wid = lax.axis_index("s") * info.num_cores + lax.axis_index("c")
        base = wid * b_per_w
        pltpu.sync_copy(idx_hbm.at[pl.ds(base, b_per_w)], idx_v)
        pltpu.async_copy(table_hbm.at[idx_v], rows_v, sem).wait()
        pltpu.sync_copy(rows_v, out_hbm.at[pl.ds(base, b_per_w)])

    return gather_k


_sc_gather = _make_sc_gather()


def _make_sc_gather_t():
    info = plsc.get_sparse_core_info()
    nw = info.num_cores * info.num_subcores
    f_per_w = D // nw
    mesh = plsc.VectorSubcoreMesh(core_axis_name="c", subcore_axis_name="s")

    @functools.partial(
        pl.kernel, mesh=mesh,
        compiler_params=pltpu.CompilerParams(
            use_tc_tiling_on_sc=False, needs_layout_passes=False),
        out_type=jax.ShapeDtypeStruct((D, B), jnp.float32),
        scratch_types=[
            pltpu.VMEM((B,), jnp.int32),
            pltpu.VMEM((f_per_w * K,), jnp.float32),
            pltpu.VMEM((f_per_w, B), jnp.float32),
        ],
    )
    def gather_t(tablet_hbm, idx_hbm, out_hbm, idx_v, tbl_v, out_v):
        wid = lax.axis_index("s") * info.num_cores + lax.axis_index("c")
        base = wid * f_per_w
        pltpu.sync_copy(tablet_hbm.at[pl.ds(base * K, f_per_w * K)], tbl_v)
        pltpu.sync_copy(idx_hbm, idx_v)
        kk = jnp.full((16,), K, jnp.int32)

        @plsc.parallel_loop(0, B // 16, unroll=16)
        def body(g):
            o = g * 16
            iv = idx_v[pl.ds(o, 16)]
            v0 = plsc.load_gather(tbl_v, [iv])
            v1 = plsc.load_gather(tbl_v, [iv + kk])
            out_v[0, pl.ds(o, 16)] = v0
            out_v[1, pl.ds(o, 16)] = v1
        pltpu.sync_copy(out_v, out_hbm.at[pl.ds(base, f_per_w)])

    return gather_t


_sc_gather_t = _make_sc_gather_t()


def kernel(x, centroids):
    assign = _assign(x.T, centroids.T)
    gathered = _sc_gather_t(centroids.T.reshape(D * K), assign).T
    return (assign, gathered)


# speedup vs baseline: 1.0622x; 1.0157x over previous
import functools

import jax
import jax.numpy as jnp
from jax import lax
from jax.experimental import pallas as pl
from jax.experimental.pallas import tpu as pltpu
from jax.experimental.pallas import tpu_sc as plsc

K = 1024
D = 64
B = 8192
BM = 2048
NB = B // BM


def _assign_body(xt_ref, ct_ref, out_ref):
    xt = xt_ref[...]
    ct = ct_ref[...]
    cc = jnp.sum(ct * ct, axis=0, keepdims=True).T
    xx = jnp.sum(xt * xt, axis=0, keepdims=True)
    xc2 = lax.dot_general(
        ct * -2.0, xt,
        dimension_numbers=(((0,), (0,)), ((), ())),
        preferred_element_type=jnp.float32,
    )
    dists = xx + xc2 + cc
    minval = jnp.min(dists, axis=0, keepdims=True)
    ids = lax.broadcasted_iota(jnp.int32, dists.shape, 0).astype(jnp.float32)
    amin = jnp.min(jnp.where(dists == minval, ids, float(K)), axis=0)
    out_ref[...] = amin.astype(jnp.int32)


def _assign(xt, ct):
    return pl.pallas_call(
        _assign_body,
        grid=(NB,),
        in_specs=[
            pl.BlockSpec((D, BM), lambda i: (0, i)),
            pl.BlockSpec((D, K), lambda i: (0, 0)),
        ],
        out_specs=pl.BlockSpec((BM,), lambda i: (i,)),
        out_shape=jax.ShapeDtypeStruct((B,), jnp.int32),
    )(xt, ct)


def _make_sc_gather():
    info = plsc.get_sparse_core_info()
    nw = info.num_cores * info.num_subcores
    b_per_w = B // nw
    mesh = plsc.VectorSubcoreMesh(core_axis_name="c", subcore_axis_name="s")

    @functools.partial(
        pl.kernel, mesh=mesh,
        compiler_params=pltpu.CompilerParams(use_tc_tiling_on_sc=False),
        out_type=jax.ShapeDtypeStruct((B, D), jnp.float32),
        scratch_types=[
            pltpu.VMEM((b_per_w,), jnp.int32),
            pltpu.VMEM((b_per_w, D), jnp.float32),
            pltpu.SemaphoreType.DMA,
        ],
    )
    def gather_k(table_hbm, idx_hbm, out_hbm, idx_v, rows_v, sem):
        wid = lax.axis_index("s") * info.num_cores + lax.axis_index("c")
        base = wid * b_per_w
        pltpu.sync_copy(idx_hbm.at[pl.ds(base, b_per_w)], idx_v)
        pltpu.async_copy(table_hbm.at[idx_v], rows_v, sem).wait()
        pltpu.sync_copy(rows_v, out_hbm.at[pl.ds(base, b_per_w)])

    return gather_k


_sc_gather = _make_sc_gather()


def _make_sc_gather_t():
    info = plsc.get_sparse_core_info()
    nw = info.num_cores * info.num_subcores
    f_per_w = D // nw
    mesh = plsc.VectorSubcoreMesh(core_axis_name="c", subcore_axis_name="s")

    @functools.partial(
        pl.kernel, mesh=mesh,
        compiler_params=pltpu.CompilerParams(
            use_tc_tiling_on_sc=False, needs_layout_passes=False),
        out_type=jax.ShapeDtypeStruct((D, B), jnp.float32),
        scratch_types=[
            pltpu.VMEM((B,), jnp.int32),
            pltpu.VMEM((f_per_w * K,), jnp.float32),
            pltpu.VMEM((f_per_w, B), jnp.float32),
        ],
    )
    def gather_t(tablet_hbm, idx_hbm, out_hbm, idx_v, tbl_v, out_v):
        wid = lax.axis_index("s") * info.num_cores + lax.axis_index("c")
        base = wid * f_per_w
        pltpu.sync_copy(tablet_hbm.at[pl.ds(base * K, f_per_w * K)], tbl_v)
        pltpu.sync_copy(idx_hbm, idx_v)
        kk = jnp.full((16,), K, jnp.int32)

        @plsc.parallel_loop(0, B // 16, unroll=16)
        def body(g):
            o = g * 16
            iv = idx_v[pl.ds(o, 16)]
            v0 = plsc.load_gather(tbl_v, [iv])
            v1 = plsc.load_gather(tbl_v, [iv + kk])
            out_v[0, pl.ds(o, 16)] = v0
            out_v[1, pl.ds(o, 16)] = v1
        pltpu.sync_copy(out_v, out_hbm.at[pl.ds(base, f_per_w)])

    return gather_t


_sc_gather_t = _make_sc_gather_t()


def kernel(x, centroids):
    assign = _assign(x.T, centroids.T)
    gathered = _sc_gather_t(centroids.T.reshape(D * K), assign).T
    return (assign, gathered)
